# Initial kernel scaffold; baseline (speedup 1.0000x reference)
import jax
import jax.numpy as jnp
from jax.experimental import pallas as pl


def kernel(x, src, tgt, Mtgt, W0, b0, W1, b1, W2, b2):
    raise NotImplementedError("write your pallas kernel here")


# speedup vs baseline: 6.4165x; 6.4165x over previous
import functools

import jax
import jax.numpy as jnp
from jax import lax
from jax.experimental import pallas as pl
from jax.experimental.pallas import tpu as pltpu
from jax.experimental.pallas import tpu_sc as plsc

N = 10000
E = 320000
NFEAT = 128
NHID = 128
NCLASS = 64

NC = 2
NS = 16
NW = NC * NS
K = 128
CHUNKS_PER_W = 79
E_PAD = NW * CHUNKS_PER_W * K
PAD = E_PAD - E
NACC = 10112
ROWS_PER_SUB = NACC // NS


def _sc_body(D, h_hbm, src_hbm, tgt_hbm, zeros_hbm, out_hbm,
             src_v, tgt_v, rows_v, acc_sh, sem):
    c = lax.axis_index("c")
    s = lax.axis_index("s")
    wid = c * NS + s

    r0 = pl.multiple_of(s * ROWS_PER_SUB, 8)
    pltpu.sync_copy(zeros_hbm.at[pl.ds(r0, ROWS_PER_SUB)],
                    acc_sh.at[pl.ds(r0, ROWS_PER_SUB)])
    plsc.subcore_barrier()

    def chunk(j, carry):
        base = pl.multiple_of((wid * CHUNKS_PER_W + j) * K, K)
        pltpu.sync_copy(src_hbm.at[pl.ds(base, K)], src_v)
        pltpu.sync_copy(tgt_hbm.at[pl.ds(base, K)], tgt_v)
        pltpu.async_copy(h_hbm.at[src_v], rows_v, sem).wait()
        pltpu.sync_copy(rows_v, acc_sh.at[tgt_v], add=True)
        return carry

    lax.fori_loop(0, CHUNKS_PER_W, chunk, 0)
    plsc.subcore_barrier()

    pltpu.sync_copy(acc_sh.at[pl.ds(r0, ROWS_PER_SUB)],
                    out_hbm.at[c, pl.ds(r0, ROWS_PER_SUB)])


def _make_sc(D):
    mesh = plsc.VectorSubcoreMesh(core_axis_name="c", subcore_axis_name="s")
    return pl.kernel(
        functools.partial(_sc_body, D),
        out_type=jax.ShapeDtypeStruct((NC, NACC, D), jnp.float32),
        mesh=mesh,
        scratch_types=[
            pltpu.VMEM((K,), jnp.int32),
            pltpu.VMEM((K,), jnp.int32),
            pltpu.VMEM((K, D), jnp.float32),
            pltpu.VMEM_SHARED((NACC, D), jnp.float32),
            pltpu.SemaphoreType.DMA,
        ],
        compiler_params=pltpu.CompilerParams(use_tc_tiling_on_sc=(D == 128)),
        name=f"gcn_edge_agg_{D}",
    )


_sc128 = _make_sc(128)
_sc64 = _make_sc(64)


_BLK = 1000


def _mm_body(x_ref, w_ref, b_ref, o_ref):
    o_ref[...] = jnp.dot(x_ref[...], w_ref[...],
                         preferred_element_type=jnp.float32) + b_ref[...]


def _mm(x, W, b):
    m, din = x.shape
    dout = W.shape[1]
    return pl.pallas_call(
        _mm_body,
        grid=(m // _BLK,),
        in_specs=[
            pl.BlockSpec((_BLK, din), lambda i: (i, 0)),
            pl.BlockSpec((din, dout), lambda i: (0, 0)),
            pl.BlockSpec((1, dout), lambda i: (0, 0)),
        ],
        out_specs=pl.BlockSpec((_BLK, dout), lambda i: (i, 0)),
        out_shape=jax.ShapeDtypeStruct((m, dout), jnp.float32),
    )(x, W, b.reshape(1, dout))


def _combine_mm_body(p_ref, mt_ref, w_ref, b_ref, t_ref, h_ref):
    t = jax.nn.relu(mt_ref[...] * (p_ref[0] + p_ref[1]))
    t_ref[...] = t
    h_ref[...] = jnp.dot(t, w_ref[...],
                         preferred_element_type=jnp.float32) + b_ref[...]


def _combine_mm(p, Mtgt, W, b):
    din = p.shape[2]
    dout = W.shape[1]
    return pl.pallas_call(
        _combine_mm_body,
        grid=(N // _BLK,),
        in_specs=[
            pl.BlockSpec((2, _BLK, din), lambda i: (0, i, 0)),
            pl.BlockSpec((_BLK, 1), lambda i: (i, 0)),
            pl.BlockSpec((din, dout), lambda i: (0, 0)),
            pl.BlockSpec((1, dout), lambda i: (0, 0)),
        ],
        out_specs=[
            pl.BlockSpec((_BLK, din), lambda i: (i, 0)),
            pl.BlockSpec((_BLK, dout), lambda i: (i, 0)),
        ],
        out_shape=[
            jax.ShapeDtypeStruct((N, din), jnp.float32),
            jax.ShapeDtypeStruct((N, dout), jnp.float32),
        ],
    )(p, Mtgt, W, b.reshape(1, dout))


def _combine_res_mm_body(p_ref, mt_ref, r_ref, w_ref, b_ref, h_ref):
    t = jax.nn.relu(mt_ref[...] * (p_ref[0] + p_ref[1])) + r_ref[...]
    h_ref[...] = jnp.dot(t, w_ref[...],
                         preferred_element_type=jnp.float32) + b_ref[...]


def _combine_res_mm(p, Mtgt, r, W, b):
    din = p.shape[2]
    dout = W.shape[1]
    return pl.pallas_call(
        _combine_res_mm_body,
        grid=(N // _BLK,),
        in_specs=[
            pl.BlockSpec((2, _BLK, din), lambda i: (0, i, 0)),
            pl.BlockSpec((_BLK, 1), lambda i: (i, 0)),
            pl.BlockSpec((_BLK, din), lambda i: (i, 0)),
            pl.BlockSpec((din, dout), lambda i: (0, 0)),
            pl.BlockSpec((1, dout), lambda i: (0, 0)),
        ],
        out_specs=pl.BlockSpec((_BLK, dout), lambda i: (i, 0)),
        out_shape=jax.ShapeDtypeStruct((N, dout), jnp.float32),
    )(p, Mtgt, r, W, b.reshape(1, dout))


def _final_body(p_ref, mt_ref, o_ref):
    y = mt_ref[...] * (p_ref[0] + p_ref[1])
    m = jnp.max(y, axis=1, keepdims=True)
    lse = m + jnp.log(jnp.sum(jnp.exp(y - m), axis=1, keepdims=True))
    o_ref[...] = y - lse


def _final(p, Mtgt):
    d = p.shape[2]
    return pl.pallas_call(
        _final_body,
        grid=(N // _BLK,),
        in_specs=[
            pl.BlockSpec((2, _BLK, d), lambda i: (0, i, 0)),
            pl.BlockSpec((_BLK, 1), lambda i: (i, 0)),
        ],
        out_specs=pl.BlockSpec((_BLK, d), lambda i: (i, 0)),
        out_shape=jax.ShapeDtypeStruct((N, d), jnp.float32),
    )(p, Mtgt)


def kernel(x, src, tgt, Mtgt, W0, b0, W1, b1, W2, b2):
    pad_i = jnp.arange(PAD, dtype=jnp.int32)
    src_p = jnp.concatenate([src, pad_i % N])
    tgt_p = jnp.concatenate([tgt, N + (pad_i % 16)])
    zeros128 = jnp.zeros((NACC, 128), jnp.float32)
    zeros64 = jnp.zeros((NACC, 64), jnp.float32)

    h0 = _mm(x, W0, b0)
    p0 = _sc128(h0, src_p, tgt_p, zeros128)
    t1, h1 = _combine_mm(p0, Mtgt, W1, b1)
    p1 = _sc128(h1, src_p, tgt_p, zeros128)
    h2 = _combine_res_mm(p1, Mtgt, t1, W2, b2)
    p2 = _sc64(h2, src_p, tgt_p, zeros64)
    return _final(p2, Mtgt)


# speedup vs baseline: 10.8169x; 1.6858x over previous
import functools

import jax
import jax.numpy as jnp
from jax import lax
from jax.experimental import pallas as pl
from jax.experimental.pallas import tpu as pltpu
from jax.experimental.pallas import tpu_sc as plsc

N = 10000
E = 320000
NFEAT = 128
NHID = 128
NCLASS = 64

NC = 2
NS = 16
K = 128
NROWS = 2560
ROWS_PER_W = NROWS // NS
E_PAD = NROWS * K
PAD = E_PAD - E
NG = ROWS_PER_W // 2
NACC = 10112
ROWS_PER_SUB = NACC // NS


def _sc_body(D, h_hbm, src_hbm, tgt_hbm, zeros_hbm, out_hbm,
             src_v, tgt_v, rows_v, acc_sh, gsem, ssem):
    c = lax.axis_index("c")
    s = lax.axis_index("s")

    i0 = pl.multiple_of(s * ROWS_PER_W, 8)
    pltpu.sync_copy(src_hbm.at[c, pl.ds(i0, ROWS_PER_W)], src_v)
    pltpu.sync_copy(tgt_hbm.at[pl.ds(i0, ROWS_PER_W)], tgt_v)

    r0 = pl.multiple_of(s * ROWS_PER_SUB, 8)
    pltpu.sync_copy(zeros_hbm.at[pl.ds(r0, ROWS_PER_SUB)],
                    acc_sh.at[pl.ds(r0, ROWS_PER_SUB)])
    plsc.subcore_barrier()

    def gather(j, slot):
        pltpu.async_copy(h_hbm.at[src_v.at[j]], rows_v.at[slot],
                         gsem.at[slot])

    def scatter(j, slot):
        pltpu.async_copy(rows_v.at[slot], acc_sh.at[tgt_v.at[j]],
                         ssem.at[slot], add=True)

    def wait_g(slot):
        pltpu.make_async_copy(h_hbm.at[src_v.at[0]], rows_v.at[slot],
                              gsem.at[slot]).wait()

    def wait_s(slot):
        pltpu.make_async_copy(rows_v.at[slot], acc_sh.at[tgt_v.at[0]],
                              ssem.at[slot]).wait()

    def half(g, sl0, sl1, osl0, osl1, wait_other_first):
        wait_g(sl0)
        wait_g(sl1)
        scatter(2 * g, sl0)
        scatter(2 * g + 1, sl1)

        @pl.when(g + 1 < NG)
        def _():
            if wait_other_first is None:
                wait_s(osl0)
                wait_s(osl1)
            else:
                @pl.when(wait_other_first)
                def _():
                    wait_s(osl0)
                    wait_s(osl1)
            gather(2 * g + 2, osl0)
            gather(2 * g + 3, osl1)

    gather(0, 0)
    gather(1, 1)

    def step(gg, carry):
        half(2 * gg, 0, 1, 2, 3, gg > 0)
        half(2 * gg + 1, 2, 3, 0, 1, None)
        return carry

    lax.fori_loop(0, NG // 2, step, 0)
    wait_s(0)
    wait_s(1)
    wait_s(2)
    wait_s(3)
    plsc.subcore_barrier()

    pltpu.sync_copy(acc_sh.at[pl.ds(r0, ROWS_PER_SUB)],
                    out_hbm.at[c, pl.ds(r0, ROWS_PER_SUB)])


def _make_sc(D):
    Dh = D // 2
    mesh = plsc.VectorSubcoreMesh(core_axis_name="c", subcore_axis_name="s")
    return pl.kernel(
        functools.partial(_sc_body, D),
        out_type=jax.ShapeDtypeStruct((NC, NACC, Dh), jnp.float32),
        mesh=mesh,
        scratch_types=[
            pltpu.VMEM((ROWS_PER_W, K), jnp.int32),
            pltpu.VMEM((ROWS_PER_W, K), jnp.int32),
            pltpu.VMEM((4, K, Dh), jnp.float32),
            pltpu.VMEM_SHARED((NACC, Dh), jnp.float32),
            pltpu.SemaphoreType.DMA((4,)),
            pltpu.SemaphoreType.DMA((4,)),
        ],
        compiler_params=pltpu.CompilerParams(use_tc_tiling_on_sc=False),
        name=f"gcn_edge_agg_{D}",
    )


_sc128 = _make_sc(128)
_sc64 = _make_sc(64)


_BLK = 1000


def _mm_body(dh, x_ref, w_ref, b_ref, o_ref):
    h = jnp.dot(x_ref[...], w_ref[...],
                preferred_element_type=jnp.float32) + b_ref[...]
    o_ref[0] = h[:, :dh]
    o_ref[1] = h[:, dh:]


def _mm(x, W, b):
    m, din = x.shape
    dout = W.shape[1]
    return pl.pallas_call(
        functools.partial(_mm_body, dout // 2),
        grid=(m // _BLK,),
        in_specs=[
            pl.BlockSpec((_BLK, din), lambda i: (i, 0)),
            pl.BlockSpec((din, dout), lambda i: (0, 0)),
            pl.BlockSpec((1, dout), lambda i: (0, 0)),
        ],
        out_specs=pl.BlockSpec((2, _BLK, dout // 2), lambda i: (0, i, 0)),
        out_shape=jax.ShapeDtypeStruct((2, m, dout // 2), jnp.float32),
    )(x, W, b.reshape(1, dout))


def _agg(p_ref, mt_ref):
    return mt_ref[...] * jnp.concatenate([p_ref[0], p_ref[1]], axis=1)


def _combine_mm_body(dh, p_ref, mt_ref, w_ref, b_ref, t_ref, h_ref):
    t = jax.nn.relu(_agg(p_ref, mt_ref))
    t_ref[...] = t
    h = jnp.dot(t, w_ref[...], preferred_element_type=jnp.float32) + b_ref[...]
    h_ref[0] = h[:, :dh]
    h_ref[1] = h[:, dh:]


def _combine_mm(p, Mtgt, W, b):
    din = 2 * p.shape[2]
    dout = W.shape[1]
    return pl.pallas_call(
        functools.partial(_combine_mm_body, dout // 2),
        grid=(N // _BLK,),
        in_specs=[
            pl.BlockSpec((2, _BLK, din // 2), lambda i: (0, i, 0)),
            pl.BlockSpec((_BLK, 1), lambda i: (i, 0)),
            pl.BlockSpec((din, dout), lambda i: (0, 0)),
            pl.BlockSpec((1, dout), lambda i: (0, 0)),
        ],
        out_specs=[
            pl.BlockSpec((_BLK, din), lambda i: (i, 0)),
            pl.BlockSpec((2, _BLK, dout // 2), lambda i: (0, i, 0)),
        ],
        out_shape=[
            jax.ShapeDtypeStruct((N, din), jnp.float32),
            jax.ShapeDtypeStruct((2, N, dout // 2), jnp.float32),
        ],
    )(p, Mtgt, W, b.reshape(1, dout))


def _combine_res_mm_body(dh, p_ref, mt_ref, r_ref, w_ref, b_ref, h_ref):
    t = jax.nn.relu(_agg(p_ref, mt_ref)) + r_ref[...]
    h = jnp.dot(t, w_ref[...], preferred_element_type=jnp.float32) + b_ref[...]
    h_ref[0] = h[:, :dh]
    h_ref[1] = h[:, dh:]


def _combine_res_mm(p, Mtgt, r, W, b):
    din = 2 * p.shape[2]
    dout = W.shape[1]
    return pl.pallas_call(
        functools.partial(_combine_res_mm_body, dout // 2),
        grid=(N // _BLK,),
        in_specs=[
            pl.BlockSpec((2, _BLK, din // 2), lambda i: (0, i, 0)),
            pl.BlockSpec((_BLK, 1), lambda i: (i, 0)),
            pl.BlockSpec((_BLK, din), lambda i: (i, 0)),
            pl.BlockSpec((din, dout), lambda i: (0, 0)),
            pl.BlockSpec((1, dout), lambda i: (0, 0)),
        ],
        out_specs=pl.BlockSpec((2, _BLK, dout // 2), lambda i: (0, i, 0)),
        out_shape=jax.ShapeDtypeStruct((2, N, dout // 2), jnp.float32),
    )(p, Mtgt, r, W, b.reshape(1, dout))


def _final_body(p_ref, mt_ref, o_ref):
    y = _agg(p_ref, mt_ref)
    m = jnp.max(y, axis=1, keepdims=True)
    lse = m + jnp.log(jnp.sum(jnp.exp(y - m), axis=1, keepdims=True))
    o_ref[...] = y - lse


def _final(p, Mtgt):
    d = 2 * p.shape[2]
    return pl.pallas_call(
        _final_body,
        grid=(N // _BLK,),
        in_specs=[
            pl.BlockSpec((2, _BLK, d // 2), lambda i: (0, i, 0)),
            pl.BlockSpec((_BLK, 1), lambda i: (i, 0)),
        ],
        out_specs=pl.BlockSpec((_BLK, d), lambda i: (i, 0)),
        out_shape=jax.ShapeDtypeStruct((N, d), jnp.float32),
    )(p, Mtgt)


def kernel(x, src, tgt, Mtgt, W0, b0, W1, b1, W2, b2):
    pad_i = jnp.arange(PAD, dtype=jnp.int32)
    src_p = jnp.concatenate([src, pad_i % N]).reshape(NROWS, K)
    src2 = jnp.stack([src_p, src_p + N])
    tgt_p = jnp.concatenate([tgt, N + (pad_i % 16)]).reshape(NROWS, K)
    zeros64 = jnp.zeros((NACC, 64), jnp.float32)
    zeros32 = jnp.zeros((NACC, 32), jnp.float32)

    h0 = _mm(x, W0, b0)
    p0 = _sc128(h0.reshape(2 * N, 64), src2, tgt_p, zeros64)
    t1, h1 = _combine_mm(p0, Mtgt, W1, b1)
    p1 = _sc128(h1.reshape(2 * N, 64), src2, tgt_p, zeros64)
    h2 = _combine_res_mm(p1, Mtgt, t1, W2, b2)
    p2 = _sc64(h2.reshape(2 * N, 32), src2, tgt_p, zeros32)
    return _final(p2, Mtgt)


# speedup vs baseline: 12.4663x; 1.1525x over previous
import functools

import jax
import jax.numpy as jnp
from jax import lax
from jax.experimental import pallas as pl
from jax.experimental.pallas import tpu as pltpu
from jax.experimental.pallas import tpu_sc as plsc

N = 10000
E = 320000
NFEAT = 128
NHID = 128
NCLASS = 64

NC = 2
NS = 16
K = 128
NROWS = 2560
ROWS_PER_W = NROWS // NS
E_PAD = NROWS * K
PAD = E_PAD - E
NG = ROWS_PER_W // 2
NACC = 10112
ROWS_PER_SUB = NACC // NS


def _sc_body(Dh, h_hbm, src_hbm, tgt_hbm, zeros_hbm, out_hbm,
             src_v, tgt_v, rows_v, acc_sh, gsem, ssem):
    c = lax.axis_index("c")
    s = lax.axis_index("s")

    i0 = pl.multiple_of(s * ROWS_PER_W, 8)
    pltpu.sync_copy(src_hbm.at[c, pl.ds(i0, ROWS_PER_W)], src_v)
    pltpu.sync_copy(tgt_hbm.at[pl.ds(i0, ROWS_PER_W)], tgt_v)

    r0 = pl.multiple_of(s * ROWS_PER_SUB, 8)
    pltpu.sync_copy(zeros_hbm.at[pl.ds(r0, ROWS_PER_SUB)],
                    acc_sh.at[pl.ds(r0, ROWS_PER_SUB)])
    plsc.subcore_barrier()

    def gather(j, slot):
        pltpu.async_copy(h_hbm.at[src_v.at[j]], rows_v.at[slot],
                         gsem.at[slot])

    def scatter(j, slot):
        pltpu.async_copy(rows_v.at[slot], acc_sh.at[tgt_v.at[j]],
                         ssem.at[slot], add=True)

    def wait_g(slot):
        pltpu.make_async_copy(h_hbm.at[src_v.at[0]], rows_v.at[slot],
                              gsem.at[slot]).wait()

    def wait_s(slot):
        pltpu.make_async_copy(rows_v.at[slot], acc_sh.at[tgt_v.at[0]],
                              ssem.at[slot]).wait()

    def half(g, sl0, sl1, osl0, osl1, wait_other_first):
        wait_g(sl0)
        wait_g(sl1)
        scatter(2 * g, sl0)
        scatter(2 * g + 1, sl1)

        @pl.when(g + 1 < NG)
        def _():
            if wait_other_first is None:
                wait_s(osl0)
                wait_s(osl1)
            else:
                @pl.when(wait_other_first)
                def _():
                    wait_s(osl0)
                    wait_s(osl1)
            gather(2 * g + 2, osl0)
            gather(2 * g + 3, osl1)

    gather(0, 0)
    gather(1, 1)

    def step(gg, carry):
        half(2 * gg, 0, 1, 2, 3, gg > 0)
        half(2 * gg + 1, 2, 3, 0, 1, None)
        return carry

    lax.fori_loop(0, NG // 2, step, 0)
    wait_s(0)
    wait_s(1)
    wait_s(2)
    wait_s(3)
    plsc.subcore_barrier()

    c0 = pl.multiple_of(c * Dh, 32)
    pltpu.sync_copy(acc_sh.at[pl.ds(r0, ROWS_PER_SUB)],
                    out_hbm.at[pl.ds(r0, ROWS_PER_SUB), pl.ds(c0, Dh)])


def _make_sc(Dh, stride):
    mesh = plsc.VectorSubcoreMesh(core_axis_name="c", subcore_axis_name="s")
    return pl.kernel(
        functools.partial(_sc_body, Dh),
        out_type=jax.ShapeDtypeStruct((NACC, 128), jnp.float32),
        mesh=mesh,
        scratch_types=[
            pltpu.VMEM((ROWS_PER_W, K), jnp.int32),
            pltpu.VMEM((ROWS_PER_W, K), jnp.int32),
            pltpu.VMEM((4, K, Dh), jnp.float32),
            pltpu.VMEM_SHARED((NACC, Dh), jnp.float32),
            pltpu.SemaphoreType.DMA((4,)),
            pltpu.SemaphoreType.DMA((4,)),
        ],
        compiler_params=pltpu.CompilerParams(use_tc_tiling_on_sc=False),
        name=f"gcn_edge_agg_{Dh}x{stride}",
    )


_sc128 = _make_sc(64, 2)
_sc64 = _make_sc(32, 4)


_BLK = 1000


def _mm_body(x_ref, w_ref, b_ref, o_ref):
    o_ref[...] = jnp.dot(x_ref[...], w_ref[...],
                         preferred_element_type=jnp.float32) + b_ref[...]


def _mm(x, W, b):
    m, din = x.shape
    dout = W.shape[1]
    return pl.pallas_call(
        _mm_body,
        grid=(m // _BLK,),
        in_specs=[
            pl.BlockSpec((_BLK, din), lambda i: (i, 0)),
            pl.BlockSpec((din, dout), lambda i: (0, 0)),
            pl.BlockSpec((1, dout), lambda i: (0, 0)),
        ],
        out_specs=pl.BlockSpec((_BLK, dout), lambda i: (i, 0)),
        out_shape=jax.ShapeDtypeStruct((m, dout), jnp.float32),
    )(x, W, b.reshape(1, dout))


def _combine_mm_body(p_ref, mt_ref, w_ref, b_ref, t_ref, h_ref):
    t = jax.nn.relu(mt_ref[...] * p_ref[...])
    t_ref[...] = t
    h_ref[...] = jnp.dot(t, w_ref[...],
                         preferred_element_type=jnp.float32) + b_ref[...]


def _combine_mm(p, Mtgt, W, b):
    din = p.shape[1]
    dout = W.shape[1]
    return pl.pallas_call(
        _combine_mm_body,
        grid=(N // _BLK,),
        in_specs=[
            pl.BlockSpec((_BLK, din), lambda i: (i, 0)),
            pl.BlockSpec((_BLK, 1), lambda i: (i, 0)),
            pl.BlockSpec((din, dout), lambda i: (0, 0)),
            pl.BlockSpec((1, dout), lambda i: (0, 0)),
        ],
        out_specs=[
            pl.BlockSpec((_BLK, din), lambda i: (i, 0)),
            pl.BlockSpec((_BLK, dout), lambda i: (i, 0)),
        ],
        out_shape=[
            jax.ShapeDtypeStruct((N, din), jnp.float32),
            jax.ShapeDtypeStruct((N, dout), jnp.float32),
        ],
    )(p, Mtgt, W, b.reshape(1, dout))


def _combine_res_mm_body(p_ref, mt_ref, r_ref, w_ref, b_ref, h_ref):
    t = jax.nn.relu(mt_ref[...] * p_ref[...]) + r_ref[...]
    h_ref[...] = jnp.dot(t, w_ref[...],
                         preferred_element_type=jnp.float32) + b_ref[...]


def _combine_res_mm(p, Mtgt, r, W, b):
    din = p.shape[1]
    dout = W.shape[1]
    return pl.pallas_call(
        _combine_res_mm_body,
        grid=(N // _BLK,),
        in_specs=[
            pl.BlockSpec((_BLK, din), lambda i: (i, 0)),
            pl.BlockSpec((_BLK, 1), lambda i: (i, 0)),
            pl.BlockSpec((_BLK, din), lambda i: (i, 0)),
            pl.BlockSpec((din, dout), lambda i: (0, 0)),
            pl.BlockSpec((1, dout), lambda i: (0, 0)),
        ],
        out_specs=pl.BlockSpec((_BLK, dout), lambda i: (i, 0)),
        out_shape=jax.ShapeDtypeStruct((N, dout), jnp.float32),
    )(p, Mtgt, r, W, b.reshape(1, dout))


def _final_body(p_ref, mt_ref, o_ref):
    y = mt_ref[...] * p_ref[:, :NCLASS]
    m = jnp.max(y, axis=1, keepdims=True)
    lse = m + jnp.log(jnp.sum(jnp.exp(y - m), axis=1, keepdims=True))
    o_ref[...] = y - lse


def _final(p, Mtgt):
    return pl.pallas_call(
        _final_body,
        grid=(N // _BLK,),
        in_specs=[
            pl.BlockSpec((_BLK, 128), lambda i: (i, 0)),
            pl.BlockSpec((_BLK, 1), lambda i: (i, 0)),
        ],
        out_specs=pl.BlockSpec((_BLK, NCLASS), lambda i: (i, 0)),
        out_shape=jax.ShapeDtypeStruct((N, NCLASS), jnp.float32),
    )(p, Mtgt)


def kernel(x, src, tgt, Mtgt, W0, b0, W1, b1, W2, b2):
    pad_i = jnp.arange(PAD, dtype=jnp.int32)
    src_p = jnp.concatenate([src, pad_i % N]).reshape(NROWS, K)
    src2 = jnp.stack([2 * src_p, 2 * src_p + 1])
    src4 = jnp.stack([4 * src_p, 4 * src_p + 1])
    tgt_p = jnp.concatenate([tgt, N + (pad_i % 16)]).reshape(NROWS, K)
    zeros64 = jnp.zeros((NACC, 64), jnp.float32)
    zeros32 = jnp.zeros((NACC, 32), jnp.float32)

    W2p = jnp.pad(W2, ((0, 0), (0, 128 - NCLASS)))
    b2p = jnp.pad(b2, (0, 128 - NCLASS))

    h0 = _mm(x, W0, b0)
    p0 = _sc128(h0.reshape(2 * N, 64), src2, tgt_p, zeros64)
    t1, h1 = _combine_mm(p0, Mtgt, W1, b1)
    p1 = _sc128(h1.reshape(2 * N, 64), src2, tgt_p, zeros64)
    h2 = _combine_res_mm(p1, Mtgt, t1, W2p, b2p)
    p2 = _sc64(h2.reshape(4 * N, 32), src4, tgt_p, zeros32)
    return _final(p2, Mtgt)


# speedup vs baseline: 15.0607x; 1.2081x over previous
import functools

import jax
import jax.numpy as jnp
from jax import lax
from jax.experimental import pallas as pl
from jax.experimental.pallas import tpu as pltpu
from jax.experimental.pallas import tpu_sc as plsc

N = 10000
E = 320000
NFEAT = 128
NHID = 128
NCLASS = 64

NC = 2
NS = 16
K = 128
NROWS = 2560
ROWS_PER_W = NROWS // NS
E_PAD = NROWS * K
PAD = E_PAD - E
NG = ROWS_PER_W // 2
NACC = 10112
ROWS_PER_SUB = NACC // NS


def _sc_body(Dh, h_hbm, src_hbm, tgt_hbm, zeros_hbm, out_hbm,
             src_v, tgt_v, rows_v, acc_sh, gsem, ssem):
    c = lax.axis_index("c")
    s = lax.axis_index("s")

    i0 = pl.multiple_of(s * ROWS_PER_W, 8)
    pltpu.sync_copy(src_hbm.at[c, pl.ds(i0, ROWS_PER_W)], src_v)
    pltpu.sync_copy(tgt_hbm.at[pl.ds(i0, ROWS_PER_W)], tgt_v)

    r0 = pl.multiple_of(s * ROWS_PER_SUB, 8)
    pltpu.sync_copy(zeros_hbm.at[pl.ds(r0, ROWS_PER_SUB)],
                    acc_sh.at[pl.ds(r0, ROWS_PER_SUB)])
    plsc.subcore_barrier()

    def gather(j, slot):
        pltpu.async_copy(h_hbm.at[src_v.at[j]], rows_v.at[slot],
                         gsem.at[slot])

    def scatter(j, slot):
        pltpu.async_copy(rows_v.at[slot], acc_sh.at[tgt_v.at[j]],
                         ssem.at[slot], add=True)

    def wait_g(slot):
        pltpu.make_async_copy(h_hbm.at[src_v.at[0]], rows_v.at[slot],
                              gsem.at[slot]).wait()

    def wait_s(slot):
        pltpu.make_async_copy(rows_v.at[slot], acc_sh.at[tgt_v.at[0]],
                              ssem.at[slot]).wait()

    R = 5
    LEAD = 3
    NITER = ROWS_PER_W // R

    gather(0, 0)
    gather(1, 1)
    gather(2, 2)

    def chunk_body(j, r, drain, refill):
        p = r
        q = (r + R - 2) % R
        wait_g(p)
        scatter(j, p)
        if drain:
            wait_s(q)
        if refill:
            gather(j + LEAD, q)

    for r in range(R):
        chunk_body(r, r, drain=(r >= 2), refill=True)

    def step(i, carry):
        for r in range(R):
            chunk_body(R * i + r, r, drain=True, refill=True)
        return carry

    lax.fori_loop(1, NITER - 1, step, 0)

    for r in range(R):
        j = R * (NITER - 1) + r
        chunk_body(j, r, drain=(j + LEAD < ROWS_PER_W),
                   refill=(j + LEAD < ROWS_PER_W))
    for r in range(R):
        wait_s(r)
    plsc.subcore_barrier()

    c0 = pl.multiple_of(c * Dh, 32)
    pltpu.sync_copy(acc_sh.at[pl.ds(r0, ROWS_PER_SUB)],
                    out_hbm.at[pl.ds(r0, ROWS_PER_SUB), pl.ds(c0, Dh)])


def _make_sc(Dh, stride):
    mesh = plsc.VectorSubcoreMesh(core_axis_name="c", subcore_axis_name="s")
    return pl.kernel(
        functools.partial(_sc_body, Dh),
        out_type=jax.ShapeDtypeStruct((NACC, 128), jnp.float32),
        mesh=mesh,
        scratch_types=[
            pltpu.VMEM((ROWS_PER_W, K), jnp.int32),
            pltpu.VMEM((ROWS_PER_W, K), jnp.int32),
            pltpu.VMEM((5, K, Dh), jnp.float32),
            pltpu.VMEM_SHARED((NACC, Dh), jnp.float32),
            pltpu.SemaphoreType.DMA((5,)),
            pltpu.SemaphoreType.DMA((5,)),
        ],
        compiler_params=pltpu.CompilerParams(use_tc_tiling_on_sc=False),
        name=f"gcn_edge_agg_{Dh}x{stride}",
    )


_sc128 = _make_sc(64, 2)
_sc64 = _make_sc(32, 4)


_BLK = 1000


def _mm_body(x_ref, w_ref, b_ref, o_ref):
    o_ref[...] = jnp.dot(x_ref[...], w_ref[...],
                         preferred_element_type=jnp.float32) + b_ref[...]


def _mm(x, W, b):
    m, din = x.shape
    dout = W.shape[1]
    return pl.pallas_call(
        _mm_body,
        grid=(m // _BLK,),
        in_specs=[
            pl.BlockSpec((_BLK, din), lambda i: (i, 0)),
            pl.BlockSpec((din, dout), lambda i: (0, 0)),
            pl.BlockSpec((1, dout), lambda i: (0, 0)),
        ],
        out_specs=pl.BlockSpec((_BLK, dout), lambda i: (i, 0)),
        out_shape=jax.ShapeDtypeStruct((m, dout), jnp.float32),
    )(x, W, b.reshape(1, dout))


def _combine_mm_body(p_ref, mt_ref, w_ref, b_ref, t_ref, h_ref):
    t = jax.nn.relu(mt_ref[...] * p_ref[...])
    t_ref[...] = t
    h_ref[...] = jnp.dot(t, w_ref[...],
                         preferred_element_type=jnp.float32) + b_ref[...]


def _combine_mm(p, Mtgt, W, b):
    din = p.shape[1]
    dout = W.shape[1]
    return pl.pallas_call(
        _combine_mm_body,
        grid=(N // _BLK,),
        in_specs=[
            pl.BlockSpec((_BLK, din), lambda i: (i, 0)),
            pl.BlockSpec((_BLK, 1), lambda i: (i, 0)),
            pl.BlockSpec((din, dout), lambda i: (0, 0)),
            pl.BlockSpec((1, dout), lambda i: (0, 0)),
        ],
        out_specs=[
            pl.BlockSpec((_BLK, din), lambda i: (i, 0)),
            pl.BlockSpec((_BLK, dout), lambda i: (i, 0)),
        ],
        out_shape=[
            jax.ShapeDtypeStruct((N, din), jnp.float32),
            jax.ShapeDtypeStruct((N, dout), jnp.float32),
        ],
    )(p, Mtgt, W, b.reshape(1, dout))


def _combine_res_mm_body(p_ref, mt_ref, r_ref, w_ref, b_ref, h_ref):
    t = jax.nn.relu(mt_ref[...] * p_ref[...]) + r_ref[...]
    h_ref[...] = jnp.dot(t, w_ref[...],
                         preferred_element_type=jnp.float32) + b_ref[...]


def _combine_res_mm(p, Mtgt, r, W, b):
    din = p.shape[1]
    dout = W.shape[1]
    return pl.pallas_call(
        _combine_res_mm_body,
        grid=(N // _BLK,),
        in_specs=[
            pl.BlockSpec((_BLK, din), lambda i: (i, 0)),
            pl.BlockSpec((_BLK, 1), lambda i: (i, 0)),
            pl.BlockSpec((_BLK, din), lambda i: (i, 0)),
            pl.BlockSpec((din, dout), lambda i: (0, 0)),
            pl.BlockSpec((1, dout), lambda i: (0, 0)),
        ],
        out_specs=pl.BlockSpec((_BLK, dout), lambda i: (i, 0)),
        out_shape=jax.ShapeDtypeStruct((N, dout), jnp.float32),
    )(p, Mtgt, r, W, b.reshape(1, dout))


def _final_body(p_ref, mt_ref, o_ref):
    y = mt_ref[...] * p_ref[:, :NCLASS]
    m = jnp.max(y, axis=1, keepdims=True)
    lse = m + jnp.log(jnp.sum(jnp.exp(y - m), axis=1, keepdims=True))
    o_ref[...] = y - lse


def _final(p, Mtgt):
    return pl.pallas_call(
        _final_body,
        grid=(N // _BLK,),
        in_specs=[
            pl.BlockSpec((_BLK, 128), lambda i: (i, 0)),
            pl.BlockSpec((_BLK, 1), lambda i: (i, 0)),
        ],
        out_specs=pl.BlockSpec((_BLK, NCLASS), lambda i: (i, 0)),
        out_shape=jax.ShapeDtypeStruct((N, NCLASS), jnp.float32),
    )(p, Mtgt)


def kernel(x, src, tgt, Mtgt, W0, b0, W1, b1, W2, b2):
    pad_i = jnp.arange(PAD, dtype=jnp.int32)
    src_p = jnp.concatenate([src, pad_i % N]).reshape(NROWS, K)
    src2 = jnp.stack([2 * src_p, 2 * src_p + 1])
    src4 = jnp.stack([4 * src_p, 4 * src_p + 1])
    tgt_p = jnp.concatenate([tgt, N + (pad_i % 16)]).reshape(NROWS, K)
    zeros64 = jnp.zeros((NACC, 64), jnp.float32)
    zeros32 = jnp.zeros((NACC, 32), jnp.float32)

    W2p = jnp.pad(W2, ((0, 0), (0, 128 - NCLASS)))
    b2p = jnp.pad(b2, (0, 128 - NCLASS))

    h0 = _mm(x, W0, b0)
    p0 = _sc128(h0.reshape(2 * N, 64), src2, tgt_p, zeros64)
    t1, h1 = _combine_mm(p0, Mtgt, W1, b1)
    p1 = _sc128(h1.reshape(2 * N, 64), src2, tgt_p, zeros64)
    h2 = _combine_res_mm(p1, Mtgt, t1, W2p, b2p)
    p2 = _sc64(h2.reshape(4 * N, 32), src4, tgt_p, zeros32)
    return _final(p2, Mtgt)


# speedup vs baseline: 15.3144x; 1.0168x over previous
import functools

import jax
import jax.numpy as jnp
from jax import lax
from jax.experimental import pallas as pl
from jax.experimental.pallas import tpu as pltpu
from jax.experimental.pallas import tpu_sc as plsc

N = 10000
E = 320000
NFEAT = 128
NHID = 128
NCLASS = 64

NC = 2
NS = 16
K = 128
NROWS = 2560
ROWS_PER_W = NROWS // NS
E_PAD = NROWS * K
PAD = E_PAD - E
NG = ROWS_PER_W // 2
NACC = 10112
ROWS_PER_SUB = NACC // NS


def _sc_body(Dh, edge_split, h_hbm, src_hbm, tgt_hbm, zeros_hbm, out_hbm,
             src_v, tgt_v, rows_v, acc_sh, gsem, ssem):
    c = lax.axis_index("c")
    s = lax.axis_index("s")

    if edge_split:
        nrw = NROWS // (2 * NS)
        i0 = pl.multiple_of((c * NS + s) * nrw, 8)
        pltpu.sync_copy(src_hbm.at[pl.ds(i0, nrw)], src_v)
    else:
        nrw = ROWS_PER_W
        i0 = pl.multiple_of(s * nrw, 8)
        pltpu.sync_copy(src_hbm.at[c, pl.ds(i0, nrw)], src_v)
    pltpu.sync_copy(tgt_hbm.at[pl.ds(i0, nrw)], tgt_v)

    r0 = pl.multiple_of(s * ROWS_PER_SUB, 8)
    pltpu.sync_copy(zeros_hbm.at[pl.ds(r0, ROWS_PER_SUB)],
                    acc_sh.at[pl.ds(r0, ROWS_PER_SUB)])
    plsc.subcore_barrier()

    def gather(j, slot):
        pltpu.async_copy(h_hbm.at[src_v.at[j]], rows_v.at[slot],
                         gsem.at[slot])

    def scatter(j, slot):
        pltpu.async_copy(rows_v.at[slot], acc_sh.at[tgt_v.at[j]],
                         ssem.at[slot], add=True)

    def wait_g(slot):
        pltpu.make_async_copy(h_hbm.at[src_v.at[0]], rows_v.at[slot],
                              gsem.at[slot]).wait()

    def wait_s(slot):
        pltpu.make_async_copy(rows_v.at[slot], acc_sh.at[tgt_v.at[0]],
                              ssem.at[slot]).wait()

    R = 5
    LEAD = 3
    NITER = nrw // R

    gather(0, 0)
    gather(1, 1)
    gather(2, 2)

    def chunk_body(j, r, drain, refill):
        p = r
        q = (r + R - 2) % R
        wait_g(p)
        scatter(j, p)
        if drain:
            wait_s(q)
        if refill:
            gather(j + LEAD, q)

    for r in range(R):
        chunk_body(r, r, drain=(r >= 2), refill=True)

    def step(i, carry):
        for r in range(R):
            chunk_body(R * i + r, r, drain=True, refill=True)
        return carry

    lax.fori_loop(1, NITER - 1, step, 0)

    for r in range(R):
        j = R * (NITER - 1) + r
        chunk_body(j, r, drain=(j + LEAD < nrw),
                   refill=(j + LEAD < nrw))
    for r in range(R):
        wait_s(r)
    plsc.subcore_barrier()

    c0 = pl.multiple_of(c * Dh, 32)
    pltpu.sync_copy(acc_sh.at[pl.ds(r0, ROWS_PER_SUB)],
                    out_hbm.at[pl.ds(r0, ROWS_PER_SUB), pl.ds(c0, Dh)])


def _make_sc(Dh, stride, edge_split=False):
    nrw = NROWS // (2 * NS) if edge_split else ROWS_PER_W
    mesh = plsc.VectorSubcoreMesh(core_axis_name="c", subcore_axis_name="s")
    return pl.kernel(
        functools.partial(_sc_body, Dh, edge_split),
        out_type=jax.ShapeDtypeStruct((NACC, 128), jnp.float32),
        mesh=mesh,
        scratch_types=[
            pltpu.VMEM((nrw, K), jnp.int32),
            pltpu.VMEM((nrw, K), jnp.int32),
            pltpu.VMEM((5, K, Dh), jnp.float32),
            pltpu.VMEM_SHARED((NACC, Dh), jnp.float32),
            pltpu.SemaphoreType.DMA((5,)),
            pltpu.SemaphoreType.DMA((5,)),
        ],
        compiler_params=pltpu.CompilerParams(use_tc_tiling_on_sc=False),
        name=f"gcn_edge_agg_{Dh}x{stride}",
    )


_sc128 = _make_sc(64, 2)
_sc64 = _make_sc(64, 2, edge_split=True)


_BLK = 1000


def _mm_body(x_ref, w_ref, b_ref, o_ref):
    o_ref[...] = jnp.dot(x_ref[...], w_ref[...],
                         preferred_element_type=jnp.float32) + b_ref[...]


def _mm(x, W, b):
    m, din = x.shape
    dout = W.shape[1]
    return pl.pallas_call(
        _mm_body,
        grid=(m // _BLK,),
        in_specs=[
            pl.BlockSpec((_BLK, din), lambda i: (i, 0)),
            pl.BlockSpec((din, dout), lambda i: (0, 0)),
            pl.BlockSpec((1, dout), lambda i: (0, 0)),
        ],
        out_specs=pl.BlockSpec((_BLK, dout), lambda i: (i, 0)),
        out_shape=jax.ShapeDtypeStruct((m, dout), jnp.float32),
    )(x, W, b.reshape(1, dout))


def _combine_mm_body(p_ref, mt_ref, w_ref, b_ref, t_ref, h_ref):
    t = jax.nn.relu(mt_ref[...] * p_ref[...])
    t_ref[...] = t
    h_ref[...] = jnp.dot(t, w_ref[...],
                         preferred_element_type=jnp.float32) + b_ref[...]


def _combine_mm(p, Mtgt, W, b):
    din = p.shape[1]
    dout = W.shape[1]
    return pl.pallas_call(
        _combine_mm_body,
        grid=(N // _BLK,),
        in_specs=[
            pl.BlockSpec((_BLK, din), lambda i: (i, 0)),
            pl.BlockSpec((_BLK, 1), lambda i: (i, 0)),
            pl.BlockSpec((din, dout), lambda i: (0, 0)),
            pl.BlockSpec((1, dout), lambda i: (0, 0)),
        ],
        out_specs=[
            pl.BlockSpec((_BLK, din), lambda i: (i, 0)),
            pl.BlockSpec((_BLK, dout), lambda i: (i, 0)),
        ],
        out_shape=[
            jax.ShapeDtypeStruct((N, din), jnp.float32),
            jax.ShapeDtypeStruct((N, dout), jnp.float32),
        ],
    )(p, Mtgt, W, b.reshape(1, dout))


def _combine_res_mm_body(p_ref, mt_ref, r_ref, w_ref, b_ref, h_ref):
    t = jax.nn.relu(mt_ref[...] * p_ref[...]) + r_ref[...]
    h_ref[...] = jnp.dot(t, w_ref[...],
                         preferred_element_type=jnp.float32) + b_ref[...]


def _combine_res_mm(p, Mtgt, r, W, b):
    din = p.shape[1]
    dout = W.shape[1]
    return pl.pallas_call(
        _combine_res_mm_body,
        grid=(N // _BLK,),
        in_specs=[
            pl.BlockSpec((_BLK, din), lambda i: (i, 0)),
            pl.BlockSpec((_BLK, 1), lambda i: (i, 0)),
            pl.BlockSpec((_BLK, din), lambda i: (i, 0)),
            pl.BlockSpec((din, dout), lambda i: (0, 0)),
            pl.BlockSpec((1, dout), lambda i: (0, 0)),
        ],
        out_specs=pl.BlockSpec((_BLK, dout), lambda i: (i, 0)),
        out_shape=jax.ShapeDtypeStruct((N, dout), jnp.float32),
    )(p, Mtgt, r, W, b.reshape(1, dout))


def _final_body(p_ref, mt_ref, o_ref):
    y = mt_ref[...] * (p_ref[:, :NCLASS] + p_ref[:, NCLASS:])
    m = jnp.max(y, axis=1, keepdims=True)
    lse = m + jnp.log(jnp.sum(jnp.exp(y - m), axis=1, keepdims=True))
    o_ref[...] = y - lse


def _final(p, Mtgt):
    return pl.pallas_call(
        _final_body,
        grid=(N // _BLK,),
        in_specs=[
            pl.BlockSpec((_BLK, 128), lambda i: (i, 0)),
            pl.BlockSpec((_BLK, 1), lambda i: (i, 0)),
        ],
        out_specs=pl.BlockSpec((_BLK, NCLASS), lambda i: (i, 0)),
        out_shape=jax.ShapeDtypeStruct((N, NCLASS), jnp.float32),
    )(p, Mtgt)


def kernel(x, src, tgt, Mtgt, W0, b0, W1, b1, W2, b2):
    pad_i = jnp.arange(PAD, dtype=jnp.int32)
    src_p = jnp.concatenate([src, pad_i % N]).reshape(NROWS, K)
    src2 = jnp.stack([2 * src_p, 2 * src_p + 1])
    src2e = 2 * src_p
    tgt_p = jnp.concatenate([tgt, N + (pad_i % 16)]).reshape(NROWS, K)
    zeros64 = jnp.zeros((NACC, 64), jnp.float32)

    W2p = jnp.pad(W2, ((0, 0), (0, 128 - NCLASS)))
    b2p = jnp.pad(b2, (0, 128 - NCLASS))

    h0 = _mm(x, W0, b0)
    p0 = _sc128(h0.reshape(2 * N, 64), src2, tgt_p, zeros64)
    t1, h1 = _combine_mm(p0, Mtgt, W1, b1)
    p1 = _sc128(h1.reshape(2 * N, 64), src2, tgt_p, zeros64)
    h2 = _combine_res_mm(p1, Mtgt, t1, W2p, b2p)
    p2 = _sc64(h2.reshape(2 * N, 64), src2e, tgt_p, zeros64)
    return _final(p2, Mtgt)


# speedup vs baseline: 15.7795x; 1.0304x over previous
import functools

import jax
import jax.numpy as jnp
from jax import lax
from jax.experimental import pallas as pl
from jax.experimental.pallas import tpu as pltpu
from jax.experimental.pallas import tpu_sc as plsc

N = 10000
E = 320000
NFEAT = 128
NHID = 128
NCLASS = 64

NC = 2
NS = 16
K = 128
NROWS = 2560
ROWS_PER_W = NROWS // NS
E_PAD = NROWS * K
PAD = E_PAD - E
NG = ROWS_PER_W // 2
NACC = 10112
ROWS_PER_SUB = NACC // NS


def _sc_body(Dh, edge_split, h_hbm, src_hbm, tgt_hbm, zeros_hbm, out_hbm,
             src_v, tgt_v, rows_v, acc_sh, gsem, ssem):
    c = lax.axis_index("c")
    s = lax.axis_index("s")

    if edge_split:
        nrw = NROWS // (2 * NS)
        i0 = pl.multiple_of((c * NS + s) * nrw, 8)
        pltpu.sync_copy(src_hbm.at[pl.ds(i0, nrw)], src_v)
    else:
        nrw = ROWS_PER_W
        i0 = pl.multiple_of(s * nrw, 8)
        pltpu.sync_copy(src_hbm.at[c, pl.ds(i0, nrw)], src_v)
    pltpu.sync_copy(tgt_hbm.at[pl.ds(i0, nrw)], tgt_v)

    r0 = pl.multiple_of(s * ROWS_PER_SUB, 8)
    pltpu.sync_copy(zeros_hbm.at[pl.ds(r0, ROWS_PER_SUB)],
                    acc_sh.at[pl.ds(r0, ROWS_PER_SUB)])
    plsc.subcore_barrier()

    def gather(j, slot):
        pltpu.async_copy(h_hbm.at[src_v.at[j]], rows_v.at[slot],
                         gsem.at[slot])

    def scatter(j, slot):
        pltpu.async_copy(rows_v.at[slot], acc_sh.at[tgt_v.at[j]],
                         ssem.at[slot], add=True)

    def wait_g(slot):
        pltpu.make_async_copy(h_hbm.at[src_v.at[0]], rows_v.at[slot],
                              gsem.at[slot]).wait()

    def wait_s(slot):
        pltpu.make_async_copy(rows_v.at[slot], acc_sh.at[tgt_v.at[0]],
                              ssem.at[slot]).wait()

    R = 5
    LEAD = 3
    NITER = nrw // R

    gather(0, 0)
    gather(1, 1)
    gather(2, 2)

    def chunk_body(j, r, drain, refill):
        p = r
        q = (r + R - 2) % R
        wait_g(p)
        scatter(j, p)
        if drain:
            wait_s(q)
        if refill:
            gather(j + LEAD, q)

    for r in range(R):
        chunk_body(r, r, drain=(r >= 2), refill=True)

    def step(i, carry):
        for r in range(R):
            chunk_body(R * i + r, r, drain=True, refill=True)
        return carry

    lax.fori_loop(1, NITER - 1, step, 0)

    for r in range(R):
        j = R * (NITER - 1) + r
        chunk_body(j, r, drain=(j + LEAD < nrw),
                   refill=(j + LEAD < nrw))
    for r in range(R):
        wait_s(r)
    plsc.subcore_barrier()

    c0 = pl.multiple_of(c * Dh, 32)
    pltpu.sync_copy(acc_sh.at[pl.ds(r0, ROWS_PER_SUB)],
                    out_hbm.at[pl.ds(r0, ROWS_PER_SUB), pl.ds(c0, Dh)])


def _make_sc(Dh, stride, edge_split=False):
    nrw = NROWS // (2 * NS) if edge_split else ROWS_PER_W
    mesh = plsc.VectorSubcoreMesh(core_axis_name="c", subcore_axis_name="s")
    return pl.kernel(
        functools.partial(_sc_body, Dh, edge_split),
        out_type=jax.ShapeDtypeStruct((NACC, 128), jnp.float32),
        mesh=mesh,
        scratch_types=[
            pltpu.VMEM((nrw, K), jnp.int32),
            pltpu.VMEM((nrw, K), jnp.int32),
            pltpu.VMEM((5, K, Dh), jnp.float32),
            pltpu.VMEM_SHARED((NACC, Dh), jnp.float32),
            pltpu.SemaphoreType.DMA((5,)),
            pltpu.SemaphoreType.DMA((5,)),
        ],
        compiler_params=pltpu.CompilerParams(use_tc_tiling_on_sc=False),
        name=f"gcn_edge_agg_{Dh}x{stride}",
    )


_sc128 = _make_sc(64, 2)
_sc64 = _make_sc(64, 2, edge_split=True)


_BLK = 2000


def _mm_body(x_ref, w_ref, b_ref, o_ref):
    o_ref[...] = jnp.dot(x_ref[...], w_ref[...],
                         preferred_element_type=jnp.float32) + b_ref[...]


def _mm(x, W, b):
    m, din = x.shape
    dout = W.shape[1]
    return pl.pallas_call(
        _mm_body,
        grid=(m // _BLK,),
        in_specs=[
            pl.BlockSpec((_BLK, din), lambda i: (i, 0)),
            pl.BlockSpec((din, dout), lambda i: (0, 0)),
            pl.BlockSpec((1, dout), lambda i: (0, 0)),
        ],
        out_specs=pl.BlockSpec((_BLK, dout), lambda i: (i, 0)),
        out_shape=jax.ShapeDtypeStruct((m, dout), jnp.float32),
    )(x, W, b.reshape(1, dout))


def _combine_mm_body(p_ref, mt_ref, w_ref, b_ref, t_ref, h_ref):
    t = jax.nn.relu(mt_ref[...] * p_ref[...])
    t_ref[...] = t
    h_ref[...] = jnp.dot(t, w_ref[...],
                         preferred_element_type=jnp.float32) + b_ref[...]


def _combine_mm(p, Mtgt, W, b):
    din = p.shape[1]
    dout = W.shape[1]
    return pl.pallas_call(
        _combine_mm_body,
        grid=(N // _BLK,),
        in_specs=[
            pl.BlockSpec((_BLK, din), lambda i: (i, 0)),
            pl.BlockSpec((_BLK, 1), lambda i: (i, 0)),
            pl.BlockSpec((din, dout), lambda i: (0, 0)),
            pl.BlockSpec((1, dout), lambda i: (0, 0)),
        ],
        out_specs=[
            pl.BlockSpec((_BLK, din), lambda i: (i, 0)),
            pl.BlockSpec((_BLK, dout), lambda i: (i, 0)),
        ],
        out_shape=[
            jax.ShapeDtypeStruct((N, din), jnp.float32),
            jax.ShapeDtypeStruct((N, dout), jnp.float32),
        ],
    )(p, Mtgt, W, b.reshape(1, dout))


def _combine_res_mm_body(p_ref, mt_ref, r_ref, w_ref, b_ref, h_ref):
    t = jax.nn.relu(mt_ref[...] * p_ref[...]) + r_ref[...]
    h_ref[...] = jnp.dot(t, w_ref[...],
                         preferred_element_type=jnp.float32) + b_ref[...]


def _combine_res_mm(p, Mtgt, r, W, b):
    din = p.shape[1]
    dout = W.shape[1]
    return pl.pallas_call(
        _combine_res_mm_body,
        grid=(N // _BLK,),
        in_specs=[
            pl.BlockSpec((_BLK, din), lambda i: (i, 0)),
            pl.BlockSpec((_BLK, 1), lambda i: (i, 0)),
            pl.BlockSpec((_BLK, din), lambda i: (i, 0)),
            pl.BlockSpec((din, dout), lambda i: (0, 0)),
            pl.BlockSpec((1, dout), lambda i: (0, 0)),
        ],
        out_specs=pl.BlockSpec((_BLK, dout), lambda i: (i, 0)),
        out_shape=jax.ShapeDtypeStruct((N, dout), jnp.float32),
    )(p, Mtgt, r, W, b.reshape(1, dout))


def _final_body(p_ref, mt_ref, o_ref):
    y = mt_ref[...] * (p_ref[:, :NCLASS] + p_ref[:, NCLASS:])
    m = jnp.max(y, axis=1, keepdims=True)
    lse = m + jnp.log(jnp.sum(jnp.exp(y - m), axis=1, keepdims=True))
    o_ref[...] = y - lse


def _final(p, Mtgt):
    return pl.pallas_call(
        _final_body,
        grid=(N // _BLK,),
        in_specs=[
            pl.BlockSpec((_BLK, 128), lambda i: (i, 0)),
            pl.BlockSpec((_BLK, 1), lambda i: (i, 0)),
        ],
        out_specs=pl.BlockSpec((_BLK, NCLASS), lambda i: (i, 0)),
        out_shape=jax.ShapeDtypeStruct((N, NCLASS), jnp.float32),
    )(p, Mtgt)


def kernel(x, src, tgt, Mtgt, W0, b0, W1, b1, W2, b2):
    pad_i = jnp.arange(PAD, dtype=jnp.int32)
    src_p = jnp.concatenate([src, pad_i % N]).reshape(NROWS, K)
    src2 = jnp.stack([2 * src_p, 2 * src_p + 1])
    src2e = 2 * src_p
    tgt_p = jnp.concatenate([tgt, N + (pad_i % 16)]).reshape(NROWS, K)
    zeros64 = jnp.zeros((NACC, 64), jnp.float32)

    W2p = jnp.pad(W2, ((0, 0), (0, 128 - NCLASS)))
    b2p = jnp.pad(b2, (0, 128 - NCLASS))

    h0 = _mm(x, W0, b0)
    p0 = _sc128(h0.reshape(2 * N, 64), src2, tgt_p, zeros64)
    t1, h1 = _combine_mm(p0, Mtgt, W1, b1)
    p1 = _sc128(h1.reshape(2 * N, 64), src2, tgt_p, zeros64)
    h2 = _combine_res_mm(p1, Mtgt, t1, W2p, b2p)
    p2 = _sc64(h2.reshape(2 * N, 64), src2e, tgt_p, zeros64)
    return _final(p2, Mtgt)


# speedup vs baseline: 16.0671x; 1.0182x over previous
import functools

import jax
import jax.numpy as jnp
from jax import lax
from jax.experimental import pallas as pl
from jax.experimental.pallas import tpu as pltpu
from jax.experimental.pallas import tpu_sc as plsc

N = 10000
E = 320000
NFEAT = 128
NHID = 128
NCLASS = 64

NC = 2
NS = 16
K = 128
NROWS = 2560
ROWS_PER_W = NROWS // NS
E_PAD = NROWS * K
PAD = E_PAD - E
NG = ROWS_PER_W // 2
NACC = 10112
ROWS_PER_SUB = NACC // NS


def _sc_body(Dh, edge_split, h_hbm, src_hbm, tgt_hbm, zeros_hbm, out_hbm,
             src_v, tgt_v, rows_v, acc_sh, gsem, ssem):
    c = lax.axis_index("c")
    s = lax.axis_index("s")

    if edge_split:
        nrw = NROWS // (2 * NS)
        i0 = pl.multiple_of((c * NS + s) * nrw, 8)
        cp_src = pltpu.async_copy(src_hbm.at[pl.ds(i0, nrw)], src_v,
                                  gsem.at[0])
    else:
        nrw = ROWS_PER_W
        i0 = pl.multiple_of(s * nrw, 8)
        cp_src = pltpu.async_copy(src_hbm.at[c, pl.ds(i0, nrw)], src_v,
                                  gsem.at[0])
    cp_tgt = pltpu.async_copy(tgt_hbm.at[pl.ds(i0, nrw)], tgt_v, gsem.at[1])
    r0 = pl.multiple_of(s * ROWS_PER_SUB, 8)
    cp_z = pltpu.async_copy(zeros_hbm.at[pl.ds(r0, ROWS_PER_SUB)],
                            acc_sh.at[pl.ds(r0, ROWS_PER_SUB)], gsem.at[2])
    cp_src.wait()
    cp_tgt.wait()
    cp_z.wait()
    plsc.subcore_barrier()

    def gather(j, slot):
        pltpu.async_copy(h_hbm.at[src_v.at[j]], rows_v.at[slot],
                         gsem.at[slot])

    def scatter(j, slot):
        pltpu.async_copy(rows_v.at[slot], acc_sh.at[tgt_v.at[j]],
                         ssem.at[slot], add=True)

    def wait_g(slot):
        pltpu.make_async_copy(h_hbm.at[src_v.at[0]], rows_v.at[slot],
                              gsem.at[slot]).wait()

    def wait_s(slot):
        pltpu.make_async_copy(rows_v.at[slot], acc_sh.at[tgt_v.at[0]],
                              ssem.at[slot]).wait()

    R = 5
    LEAD = 3
    NITER = nrw // R

    gather(0, 0)
    gather(1, 1)
    gather(2, 2)

    def chunk_body(j, r, drain, refill):
        p = r
        q = (r + R - 2) % R
        wait_g(p)
        scatter(j, p)
        if drain:
            wait_s(q)
        if refill:
            gather(j + LEAD, q)

    for r in range(R):
        chunk_body(r, r, drain=(r >= 2), refill=True)

    def step(i, carry):
        for r in range(R):
            chunk_body(R * i + r, r, drain=True, refill=True)
        return carry

    lax.fori_loop(1, NITER - 1, step, 0)

    for r in range(R):
        j = R * (NITER - 1) + r
        chunk_body(j, r, drain=(j + LEAD < nrw),
                   refill=(j + LEAD < nrw))
    for r in range(R):
        wait_s(r)
    plsc.subcore_barrier()

    c0 = pl.multiple_of(c * Dh, 32)
    pltpu.sync_copy(acc_sh.at[pl.ds(r0, ROWS_PER_SUB)],
                    out_hbm.at[pl.ds(r0, ROWS_PER_SUB), pl.ds(c0, Dh)])


def _make_sc(Dh, stride, edge_split=False):
    nrw = NROWS // (2 * NS) if edge_split else ROWS_PER_W
    mesh = plsc.VectorSubcoreMesh(core_axis_name="c", subcore_axis_name="s")
    return pl.kernel(
        functools.partial(_sc_body, Dh, edge_split),
        out_type=jax.ShapeDtypeStruct((NACC, 128), jnp.float32),
        mesh=mesh,
        scratch_types=[
            pltpu.VMEM((nrw, K), jnp.int32),
            pltpu.VMEM((nrw, K), jnp.int32),
            pltpu.VMEM((5, K, Dh), jnp.float32),
            pltpu.VMEM_SHARED((NACC, Dh), jnp.float32),
            pltpu.SemaphoreType.DMA((5,)),
            pltpu.SemaphoreType.DMA((5,)),
        ],
        compiler_params=pltpu.CompilerParams(use_tc_tiling_on_sc=False),
        name=f"gcn_edge_agg_{Dh}x{stride}",
    )


_sc128 = _make_sc(64, 2)
_sc64 = _make_sc(64, 2, edge_split=True)


_BLK = 2000


def _mm_body(x_ref, w_ref, b_ref, o_ref):
    o_ref[...] = jnp.dot(x_ref[...], w_ref[...],
                         preferred_element_type=jnp.float32) + b_ref[...]


def _mm(x, W, b):
    m, din = x.shape
    dout = W.shape[1]
    return pl.pallas_call(
        _mm_body,
        grid=(m // _BLK,),
        in_specs=[
            pl.BlockSpec((_BLK, din), lambda i: (i, 0)),
            pl.BlockSpec((din, dout), lambda i: (0, 0)),
            pl.BlockSpec((1, dout), lambda i: (0, 0)),
        ],
        out_specs=pl.BlockSpec((_BLK, dout), lambda i: (i, 0)),
        out_shape=jax.ShapeDtypeStruct((m, dout), jnp.float32),
    )(x, W, b.reshape(1, dout))


def _combine_mm_body(p_ref, mt_ref, w_ref, b_ref, t_ref, h_ref):
    t = jax.nn.relu(mt_ref[...] * p_ref[...])
    t_ref[...] = t
    h_ref[...] = jnp.dot(t, w_ref[...],
                         preferred_element_type=jnp.float32) + b_ref[...]


def _combine_mm(p, Mtgt, W, b):
    din = p.shape[1]
    dout = W.shape[1]
    return pl.pallas_call(
        _combine_mm_body,
        grid=(N // _BLK,),
        in_specs=[
            pl.BlockSpec((_BLK, din), lambda i: (i, 0)),
            pl.BlockSpec((_BLK, 1), lambda i: (i, 0)),
            pl.BlockSpec((din, dout), lambda i: (0, 0)),
            pl.BlockSpec((1, dout), lambda i: (0, 0)),
        ],
        out_specs=[
            pl.BlockSpec((_BLK, din), lambda i: (i, 0)),
            pl.BlockSpec((_BLK, dout), lambda i: (i, 0)),
        ],
        out_shape=[
            jax.ShapeDtypeStruct((N, din), jnp.float32),
            jax.ShapeDtypeStruct((N, dout), jnp.float32),
        ],
    )(p, Mtgt, W, b.reshape(1, dout))


def _combine_res_mm_body(p_ref, mt_ref, r_ref, w_ref, b_ref, h_ref):
    t = jax.nn.relu(mt_ref[...] * p_ref[...]) + r_ref[...]
    h_ref[...] = jnp.dot(t, w_ref[...],
                         preferred_element_type=jnp.float32) + b_ref[...]


def _combine_res_mm(p, Mtgt, r, W, b):
    din = p.shape[1]
    dout = W.shape[1]
    return pl.pallas_call(
        _combine_res_mm_body,
        grid=(N // _BLK,),
        in_specs=[
            pl.BlockSpec((_BLK, din), lambda i: (i, 0)),
            pl.BlockSpec((_BLK, 1), lambda i: (i, 0)),
            pl.BlockSpec((_BLK, din), lambda i: (i, 0)),
            pl.BlockSpec((din, dout), lambda i: (0, 0)),
            pl.BlockSpec((1, dout), lambda i: (0, 0)),
        ],
        out_specs=pl.BlockSpec((_BLK, dout), lambda i: (i, 0)),
        out_shape=jax.ShapeDtypeStruct((N, dout), jnp.float32),
    )(p, Mtgt, r, W, b.reshape(1, dout))


def _final_body(p_ref, mt_ref, o_ref):
    y = mt_ref[...] * (p_ref[:, :NCLASS] + p_ref[:, NCLASS:])
    m = jnp.max(y, axis=1, keepdims=True)
    lse = m + jnp.log(jnp.sum(jnp.exp(y - m), axis=1, keepdims=True))
    o_ref[...] = y - lse


def _final(p, Mtgt):
    return pl.pallas_call(
        _final_body,
        grid=(N // _BLK,),
        in_specs=[
            pl.BlockSpec((_BLK, 128), lambda i: (i, 0)),
            pl.BlockSpec((_BLK, 1), lambda i: (i, 0)),
        ],
        out_specs=pl.BlockSpec((_BLK, NCLASS), lambda i: (i, 0)),
        out_shape=jax.ShapeDtypeStruct((N, NCLASS), jnp.float32),
    )(p, Mtgt)


def kernel(x, src, tgt, Mtgt, W0, b0, W1, b1, W2, b2):
    pad_i = jnp.arange(PAD, dtype=jnp.int32)
    src_p = jnp.concatenate([src, pad_i % N]).reshape(NROWS, K)
    src2 = jnp.stack([2 * src_p, 2 * src_p + 1])
    src2e = 2 * src_p
    tgt_p = jnp.concatenate([tgt, N + (pad_i % 16)]).reshape(NROWS, K)
    zeros64 = jnp.zeros((NACC, 64), jnp.float32)

    W2p = jnp.pad(W2, ((0, 0), (0, 128 - NCLASS)))
    b2p = jnp.pad(b2, (0, 128 - NCLASS))

    h0 = _mm(x, W0, b0)
    p0 = _sc128(h0.reshape(2 * N, 64), src2, tgt_p, zeros64)
    t1, h1 = _combine_mm(p0, Mtgt, W1, b1)
    p1 = _sc128(h1.reshape(2 * N, 64), src2, tgt_p, zeros64)
    h2 = _combine_res_mm(p1, Mtgt, t1, W2p, b2p)
    p2 = _sc64(h2.reshape(2 * N, 64), src2e, tgt_p, zeros64)
    return _final(p2, Mtgt)


# speedup vs baseline: 16.2680x; 1.0125x over previous
import functools

import jax
import jax.numpy as jnp
from jax import lax
from jax.experimental import pallas as pl
from jax.experimental.pallas import tpu as pltpu
from jax.experimental.pallas import tpu_sc as plsc

N = 10000
E = 320000
NFEAT = 128
NHID = 128
NCLASS = 64

NC = 2
NS = 16
K = 128
NROWS = 2560
ROWS_PER_W = NROWS // NS
E_PAD = NROWS * K
PAD = E_PAD - E
NG = ROWS_PER_W // 2
NACC = 10112
ROWS_PER_SUB = NACC // NS


def _sc_body(Dh, edge_split, h_hbm, src_hbm, tgt_hbm, zeros_hbm, out_hbm,
             src_v, tgt_v, rows_v, acc_sh, gsem, ssem):
    c = lax.axis_index("c")
    s = lax.axis_index("s")

    if edge_split:
        nrw = NROWS // (2 * NS)
        i0 = pl.multiple_of((c * NS + s) * nrw, 8)
        cp_src = pltpu.async_copy(src_hbm.at[pl.ds(i0, nrw)], src_v,
                                  gsem.at[0])
    else:
        nrw = ROWS_PER_W
        i0 = pl.multiple_of(s * nrw, 8)
        cp_src = pltpu.async_copy(src_hbm.at[c, pl.ds(i0, nrw)], src_v,
                                  gsem.at[0])
    cp_tgt = pltpu.async_copy(tgt_hbm.at[pl.ds(i0, nrw)], tgt_v, gsem.at[1])
    r0 = pl.multiple_of(s * ROWS_PER_SUB, 8)
    cp_z = pltpu.async_copy(zeros_hbm.at[pl.ds(r0, ROWS_PER_SUB)],
                            acc_sh.at[pl.ds(r0, ROWS_PER_SUB)], gsem.at[2])
    cp_src.wait()
    cp_tgt.wait()
    cp_z.wait()
    plsc.subcore_barrier()

    def gather(j, slot):
        pltpu.async_copy(h_hbm.at[src_v.at[j]], rows_v.at[slot],
                         gsem.at[slot])

    def scatter(j, slot):
        pltpu.async_copy(rows_v.at[slot], acc_sh.at[tgt_v.at[j]],
                         ssem.at[slot], add=True)

    def wait_g(slot):
        pltpu.make_async_copy(h_hbm.at[src_v.at[0]], rows_v.at[slot],
                              gsem.at[slot]).wait()

    def wait_s(slot):
        pltpu.make_async_copy(rows_v.at[slot], acc_sh.at[tgt_v.at[0]],
                              ssem.at[slot]).wait()

    R = 5
    LEAD = 3
    NITER = nrw // R

    gather(0, 0)
    gather(1, 1)
    gather(2, 2)

    def chunk_body(j, r, drain, refill):
        p = r
        q = (r + R - 2) % R
        wait_g(p)
        scatter(j, p)
        if drain:
            wait_s(q)
        if refill:
            gather(j + LEAD, q)

    for r in range(R):
        chunk_body(r, r, drain=(r >= 2), refill=True)

    def step(i, carry):
        for r in range(R):
            chunk_body(R * i + r, r, drain=True, refill=True)
        return carry

    lax.fori_loop(1, NITER - 1, step, 0)

    for r in range(R):
        j = R * (NITER - 1) + r
        chunk_body(j, r, drain=(j + LEAD < nrw),
                   refill=(j + LEAD < nrw))
    for r in range(R):
        wait_s(r)
    plsc.subcore_barrier()

    c0 = pl.multiple_of(c * Dh, 32)
    pltpu.sync_copy(acc_sh.at[pl.ds(r0, ROWS_PER_SUB)],
                    out_hbm.at[pl.ds(r0, ROWS_PER_SUB), pl.ds(c0, Dh)])


def _make_sc(Dh, stride, edge_split=False):
    nrw = NROWS // (2 * NS) if edge_split else ROWS_PER_W
    mesh = plsc.VectorSubcoreMesh(core_axis_name="c", subcore_axis_name="s")
    return pl.kernel(
        functools.partial(_sc_body, Dh, edge_split),
        out_type=jax.ShapeDtypeStruct((NACC, 128), jnp.float32),
        mesh=mesh,
        scratch_types=[
            pltpu.VMEM((nrw, K), jnp.int32),
            pltpu.VMEM((nrw, K), jnp.int32),
            pltpu.VMEM((5, K, Dh), jnp.float32),
            pltpu.VMEM_SHARED((NACC, Dh), jnp.float32),
            pltpu.SemaphoreType.DMA((5,)),
            pltpu.SemaphoreType.DMA((5,)),
        ],
        compiler_params=pltpu.CompilerParams(use_tc_tiling_on_sc=False),
        name=f"gcn_edge_agg_{Dh}x{stride}",
    )


_sc128 = _make_sc(64, 2)
_sc64 = _make_sc(64, 2, edge_split=True)


_BLK = 2000


def _mm_body(x_ref, w_ref, b_ref, o_ref):
    o_ref[...] = jnp.dot(x_ref[...], w_ref[...],
                         preferred_element_type=jnp.float32) + b_ref[...]


def _mm(x, W, b):
    m, din = x.shape
    dout = W.shape[1]
    return pl.pallas_call(
        _mm_body,
        grid=(m // _BLK,),
        in_specs=[
            pl.BlockSpec((_BLK, din), lambda i: (i, 0)),
            pl.BlockSpec((din, dout), lambda i: (0, 0)),
            pl.BlockSpec((1, dout), lambda i: (0, 0)),
        ],
        out_specs=pl.BlockSpec((_BLK, dout), lambda i: (i, 0)),
        out_shape=jax.ShapeDtypeStruct((m, dout), jnp.float32),
    )(x, W, b.reshape(1, dout))


def _combine_mm_body(p_ref, mt_ref, w_ref, b_ref, t_ref, h_ref):
    t = jax.nn.relu(mt_ref[...] * p_ref[...])
    t_ref[...] = t
    h_ref[...] = jnp.dot(t, w_ref[...],
                         preferred_element_type=jnp.float32) + b_ref[...]


def _combine_mm(p, Mtgt, W, b):
    din = p.shape[1]
    dout = W.shape[1]
    return pl.pallas_call(
        _combine_mm_body,
        grid=(N // _BLK,),
        in_specs=[
            pl.BlockSpec((_BLK, din), lambda i: (i, 0)),
            pl.BlockSpec((_BLK, 1), lambda i: (i, 0)),
            pl.BlockSpec((din, dout), lambda i: (0, 0)),
            pl.BlockSpec((1, dout), lambda i: (0, 0)),
        ],
        out_specs=[
            pl.BlockSpec((_BLK, din), lambda i: (i, 0)),
            pl.BlockSpec((_BLK, dout), lambda i: (i, 0)),
        ],
        out_shape=[
            jax.ShapeDtypeStruct((N, din), jnp.float32),
            jax.ShapeDtypeStruct((N, dout), jnp.float32),
        ],
    )(p, Mtgt, W, b.reshape(1, dout))


def _combine_res_mm_body(p_ref, mt_ref, r_ref, w_ref, b_ref, h_ref):
    t = jax.nn.relu(mt_ref[...] * p_ref[...]) + r_ref[...]
    h_ref[...] = jnp.dot(t, w_ref[...],
                         preferred_element_type=jnp.float32) + b_ref[...]


def _combine_res_mm(p, Mtgt, r, W, b):
    din = p.shape[1]
    dout = W.shape[1]
    return pl.pallas_call(
        _combine_res_mm_body,
        grid=(N // _BLK,),
        in_specs=[
            pl.BlockSpec((_BLK, din), lambda i: (i, 0)),
            pl.BlockSpec((_BLK, 1), lambda i: (i, 0)),
            pl.BlockSpec((_BLK, din), lambda i: (i, 0)),
            pl.BlockSpec((din, dout), lambda i: (0, 0)),
            pl.BlockSpec((1, dout), lambda i: (0, 0)),
        ],
        out_specs=pl.BlockSpec((_BLK, dout), lambda i: (i, 0)),
        out_shape=jax.ShapeDtypeStruct((N, dout), jnp.float32),
    )(p, Mtgt, r, W, b.reshape(1, dout))


def _final_body(p_ref, mt_ref, o_ref):
    y = mt_ref[...] * (p_ref[:, :NCLASS] + p_ref[:, NCLASS:])
    m = jnp.max(y, axis=1, keepdims=True)
    lse = m + jnp.log(jnp.sum(jnp.exp(y - m), axis=1, keepdims=True))
    o_ref[...] = (y - lse).T


def _final(p, Mtgt):
    return pl.pallas_call(
        _final_body,
        grid=(1,),
        in_specs=[
            pl.BlockSpec((N, 128), lambda i: (0, 0)),
            pl.BlockSpec((N, 1), lambda i: (0, 0)),
        ],
        out_specs=pl.BlockSpec((NCLASS, N), lambda i: (0, 0)),
        out_shape=jax.ShapeDtypeStruct((NCLASS, N), jnp.float32),
    )(p, Mtgt).T


def kernel(x, src, tgt, Mtgt, W0, b0, W1, b1, W2, b2):
    pad_i = jnp.arange(PAD, dtype=jnp.int32)
    src_p = jnp.concatenate([src, pad_i % N]).reshape(NROWS, K)
    src2 = jnp.stack([2 * src_p, 2 * src_p + 1])
    src2e = 2 * src_p
    tgt_p = jnp.concatenate([tgt, N + (pad_i % 16)]).reshape(NROWS, K)
    zeros64 = jnp.zeros((NACC, 64), jnp.float32)

    W2p = jnp.pad(W2, ((0, 0), (0, 128 - NCLASS)))
    b2p = jnp.pad(b2, (0, 128 - NCLASS))

    h0 = _mm(x, W0, b0)
    p0 = _sc128(h0.reshape(2 * N, 64), src2, tgt_p, zeros64)
    t1, h1 = _combine_mm(p0, Mtgt, W1, b1)
    p1 = _sc128(h1.reshape(2 * N, 64), src2, tgt_p, zeros64)
    h2 = _combine_res_mm(p1, Mtgt, t1, W2p, b2p)
    p2 = _sc64(h2.reshape(2 * N, 64), src2e, tgt_p, zeros64)
    return _final(p2, Mtgt)


# speedup vs baseline: 16.7015x; 1.0266x over previous
import functools

import jax
import jax.numpy as jnp
from jax import lax
from jax.experimental import pallas as pl
from jax.experimental.pallas import tpu as pltpu
from jax.experimental.pallas import tpu_sc as plsc

N = 10000
E = 320000
NFEAT = 128
NHID = 128
NCLASS = 64

NC = 2
NS = 16
K = 128
NROWS = 2560
ROWS_PER_W = NROWS // NS
E_PAD = NROWS * K
PAD = E_PAD - E
NG = ROWS_PER_W // 2
NACC = 10112
ROWS_PER_SUB = NACC // NS


def _sc_body(Dh, edge_split, h_hbm, src_hbm, tgt_hbm, zeros_hbm, out_hbm,
             src_v, tgt_v, rows_v, acc_sh, gsem, ssem):
    c = lax.axis_index("c")
    s = lax.axis_index("s")

    if edge_split:
        nrw = NROWS // (2 * NS)
        i0 = pl.multiple_of((c * NS + s) * nrw, 8)
        cp_src = pltpu.async_copy(src_hbm.at[pl.ds(i0, nrw)], src_v,
                                  gsem.at[0])
    else:
        nrw = ROWS_PER_W
        i0 = pl.multiple_of(s * nrw, 8)
        cp_src = pltpu.async_copy(src_hbm.at[c, pl.ds(i0, nrw)], src_v,
                                  gsem.at[0])
    cp_tgt = pltpu.async_copy(tgt_hbm.at[pl.ds(i0, nrw)], tgt_v, gsem.at[1])
    r0 = pl.multiple_of(s * ROWS_PER_SUB, 8)
    cp_z = pltpu.async_copy(zeros_hbm.at[pl.ds(r0, ROWS_PER_SUB)],
                            acc_sh.at[pl.ds(r0, ROWS_PER_SUB)], gsem.at[2])
    cp_src.wait()
    cp_tgt.wait()
    cp_z.wait()
    plsc.subcore_barrier()

    def gather(j, slot):
        pltpu.async_copy(h_hbm.at[src_v.at[j]], rows_v.at[slot],
                         gsem.at[slot])

    def scatter(j, slot):
        pltpu.async_copy(rows_v.at[slot], acc_sh.at[tgt_v.at[j]],
                         ssem.at[slot], add=True)

    def wait_g(slot):
        pltpu.make_async_copy(h_hbm.at[src_v.at[0]], rows_v.at[slot],
                              gsem.at[slot]).wait()

    def wait_s(slot):
        pltpu.make_async_copy(rows_v.at[slot], acc_sh.at[tgt_v.at[0]],
                              ssem.at[slot]).wait()

    R = 5
    DRAIN = 1
    LEAD = R - DRAIN
    NITER = nrw // R

    for r in range(LEAD):
        gather(r, r)

    def chunk_body(j, r, drain, refill):
        p = r
        q = (r + R - DRAIN) % R
        wait_g(p)
        scatter(j, p)
        if drain:
            wait_s(q)
        if refill:
            gather(j + LEAD, q)

    for r in range(R):
        chunk_body(r, r, drain=(r >= DRAIN), refill=True)

    def step(i, carry):
        for r in range(R):
            chunk_body(R * i + r, r, drain=True, refill=True)
        return carry

    lax.fori_loop(1, NITER - 1, step, 0)

    for r in range(R):
        j = R * (NITER - 1) + r
        chunk_body(j, r, drain=(j + LEAD < nrw),
                   refill=(j + LEAD < nrw))
    for r in range(R):
        wait_s(r)
    plsc.subcore_barrier()

    c0 = pl.multiple_of(c * Dh, 32)
    pltpu.sync_copy(acc_sh.at[pl.ds(r0, ROWS_PER_SUB)],
                    out_hbm.at[pl.ds(r0, ROWS_PER_SUB), pl.ds(c0, Dh)])


def _make_sc(Dh, stride, edge_split=False):
    nrw = NROWS // (2 * NS) if edge_split else ROWS_PER_W
    mesh = plsc.VectorSubcoreMesh(core_axis_name="c", subcore_axis_name="s")
    return pl.kernel(
        functools.partial(_sc_body, Dh, edge_split),
        out_type=jax.ShapeDtypeStruct((NACC, 128), jnp.float32),
        mesh=mesh,
        scratch_types=[
            pltpu.VMEM((nrw, K), jnp.int32),
            pltpu.VMEM((nrw, K), jnp.int32),
            pltpu.VMEM((5, K, Dh), jnp.float32),
            pltpu.VMEM_SHARED((NACC, Dh), jnp.float32),
            pltpu.SemaphoreType.DMA((5,)),
            pltpu.SemaphoreType.DMA((5,)),
        ],
        compiler_params=pltpu.CompilerParams(use_tc_tiling_on_sc=False),
        name=f"gcn_edge_agg_{Dh}x{stride}",
    )


_sc128 = _make_sc(64, 2)
_sc64 = _make_sc(64, 2, edge_split=True)


_BLK = 2000


def _mm_body(x_ref, w_ref, b_ref, o_ref):
    o_ref[...] = jnp.dot(x_ref[...], w_ref[...],
                         preferred_element_type=jnp.float32) + b_ref[...]


def _mm(x, W, b):
    m, din = x.shape
    dout = W.shape[1]
    return pl.pallas_call(
        _mm_body,
        grid=(m // _BLK,),
        in_specs=[
            pl.BlockSpec((_BLK, din), lambda i: (i, 0)),
            pl.BlockSpec((din, dout), lambda i: (0, 0)),
            pl.BlockSpec((1, dout), lambda i: (0, 0)),
        ],
        out_specs=pl.BlockSpec((_BLK, dout), lambda i: (i, 0)),
        out_shape=jax.ShapeDtypeStruct((m, dout), jnp.float32),
    )(x, W, b.reshape(1, dout))


def _combine_mm_body(p_ref, mt_ref, w_ref, b_ref, t_ref, h_ref):
    t = jax.nn.relu(mt_ref[...] * p_ref[...])
    t_ref[...] = t
    h_ref[...] = jnp.dot(t, w_ref[...],
                         preferred_element_type=jnp.float32) + b_ref[...]


def _combine_mm(p, Mtgt, W, b):
    din = p.shape[1]
    dout = W.shape[1]
    return pl.pallas_call(
        _combine_mm_body,
        grid=(N // _BLK,),
        in_specs=[
            pl.BlockSpec((_BLK, din), lambda i: (i, 0)),
            pl.BlockSpec((_BLK, 1), lambda i: (i, 0)),
            pl.BlockSpec((din, dout), lambda i: (0, 0)),
            pl.BlockSpec((1, dout), lambda i: (0, 0)),
        ],
        out_specs=[
            pl.BlockSpec((_BLK, din), lambda i: (i, 0)),
            pl.BlockSpec((_BLK, dout), lambda i: (i, 0)),
        ],
        out_shape=[
            jax.ShapeDtypeStruct((N, din), jnp.float32),
            jax.ShapeDtypeStruct((N, dout), jnp.float32),
        ],
    )(p, Mtgt, W, b.reshape(1, dout))


def _combine_res_mm_body(p_ref, mt_ref, r_ref, w_ref, b_ref, h_ref):
    t = jax.nn.relu(mt_ref[...] * p_ref[...]) + r_ref[...]
    h_ref[...] = jnp.dot(t, w_ref[...],
                         preferred_element_type=jnp.float32) + b_ref[...]


def _combine_res_mm(p, Mtgt, r, W, b):
    din = p.shape[1]
    dout = W.shape[1]
    return pl.pallas_call(
        _combine_res_mm_body,
        grid=(N // _BLK,),
        in_specs=[
            pl.BlockSpec((_BLK, din), lambda i: (i, 0)),
            pl.BlockSpec((_BLK, 1), lambda i: (i, 0)),
            pl.BlockSpec((_BLK, din), lambda i: (i, 0)),
            pl.BlockSpec((din, dout), lambda i: (0, 0)),
            pl.BlockSpec((1, dout), lambda i: (0, 0)),
        ],
        out_specs=pl.BlockSpec((_BLK, dout), lambda i: (i, 0)),
        out_shape=jax.ShapeDtypeStruct((N, dout), jnp.float32),
    )(p, Mtgt, r, W, b.reshape(1, dout))


def _final_body(p_ref, mt_ref, o_ref):
    y = mt_ref[...] * (p_ref[:, :NCLASS] + p_ref[:, NCLASS:])
    m = jnp.max(y, axis=1, keepdims=True)
    lse = m + jnp.log(jnp.sum(jnp.exp(y - m), axis=1, keepdims=True))
    o_ref[...] = (y - lse).T


def _final(p, Mtgt):
    return pl.pallas_call(
        _final_body,
        grid=(1,),
        in_specs=[
            pl.BlockSpec((N, 128), lambda i: (0, 0)),
            pl.BlockSpec((N, 1), lambda i: (0, 0)),
        ],
        out_specs=pl.BlockSpec((NCLASS, N), lambda i: (0, 0)),
        out_shape=jax.ShapeDtypeStruct((NCLASS, N), jnp.float32),
    )(p, Mtgt).T


def kernel(x, src, tgt, Mtgt, W0, b0, W1, b1, W2, b2):
    pad_i = jnp.arange(PAD, dtype=jnp.int32)
    src_p = jnp.concatenate([src, pad_i % N]).reshape(NROWS, K)
    src2 = jnp.stack([2 * src_p, 2 * src_p + 1])
    src2e = 2 * src_p
    tgt_p = jnp.concatenate([tgt, N + (pad_i % 16)]).reshape(NROWS, K)
    zeros64 = jnp.zeros((NACC, 64), jnp.float32)

    W2p = jnp.pad(W2, ((0, 0), (0, 128 - NCLASS)))
    b2p = jnp.pad(b2, (0, 128 - NCLASS))

    h0 = _mm(x, W0, b0)
    p0 = _sc128(h0.reshape(2 * N, 64), src2, tgt_p, zeros64)
    t1, h1 = _combine_mm(p0, Mtgt, W1, b1)
    p1 = _sc128(h1.reshape(2 * N, 64), src2, tgt_p, zeros64)
    h2 = _combine_res_mm(p1, Mtgt, t1, W2p, b2p)
    p2 = _sc64(h2.reshape(2 * N, 64), src2e, tgt_p, zeros64)
    return _final(p2, Mtgt)
